# baseline (speedup 1.0000x reference)
import functools

import jax
import jax.numpy as jnp
import numpy as np
from jax import lax
from jax.experimental import pallas as pl
from jax.experimental.pallas import tpu as pltpu
from jax.experimental.pallas import tpu_sc as plsc

B = 8
N = 16384
S = 256
NS = 64
CIN = 128
COUT = 256
EPS = 1e-5
RR = np.float32(0.32 * 0.32)
M_TOT = B * S * NS

_NCH = 4
_SCH = S // _NCH
_G_ROWS = B * _SCH * NS


def _fps_body(xyz_sb_ref, dists_in_ref, far_in_ref,
              cen_ref, dists_ref, far_ref):
    i = pl.program_id(0)
    xs = xyz_sb_ref[0:B, :]
    ys = xyz_sb_ref[B:2 * B, :]
    zs = xyz_sb_ref[2 * B:3 * B, :]
    iota = lax.broadcasted_iota(jnp.int32, (B, N), 1)

    @pl.when(i == 0)
    def _():
        dists_ref[...] = dists_in_ref[...]
        far_ref[...] = far_in_ref[...]

    far = far_ref[...]
    onehot = iota == far
    cx = jnp.sum(jnp.where(onehot, xs, 0.0), axis=1, keepdims=True)
    cy = jnp.sum(jnp.where(onehot, ys, 0.0), axis=1, keepdims=True)
    cz = jnp.sum(jnp.where(onehot, zs, 0.0), axis=1, keepdims=True)
    cen_ref[...] = jnp.concatenate([cx, cy, cz], axis=1)[None]
    dx = xs - cx
    dy = ys - cy
    dz = zs - cz
    d = dx * dx + dy * dy + dz * dz
    dmin = jnp.minimum(dists_ref[...], d)
    dists_ref[...] = dmin
    mx = jnp.max(dmin, axis=1, keepdims=True)
    cand = jnp.where(dmin == mx, iota, N)
    far_ref[...] = jnp.min(cand, axis=1, keepdims=True).astype(jnp.int32)


def _fps_chunk(xyz_sb, dists, far):
    return pl.pallas_call(
        _fps_body,
        grid=(_SCH,),
        in_specs=[
            pl.BlockSpec((3 * B, N), lambda i: (0, 0)),
            pl.BlockSpec((B, N), lambda i: (0, 0)),
            pl.BlockSpec((B, 1), lambda i: (0, 0)),
        ],
        out_specs=[
            pl.BlockSpec((1, B, 3), lambda i: (i, 0, 0)),
            pl.BlockSpec((B, N), lambda i: (0, 0)),
            pl.BlockSpec((B, 1), lambda i: (0, 0)),
        ],
        out_shape=[
            jax.ShapeDtypeStruct((_SCH, B, 3), jnp.float32),
            jax.ShapeDtypeStruct((B, N), jnp.float32),
            jax.ShapeDtypeStruct((B, 1), jnp.int32),
        ],
    )(xyz_sb, dists, far)


_RPW = 16
_W_PER_B = 4
_SUB_PER_SUPER = 16
_SUPER = 16 * _SUB_PER_SUPER


def _bq_gather_body(xyz3_hbm, cen4_hbm, ftf_hbm, out_hbm,
                    xs, ys, zs, cenv, rowbuf, gidx_a, gidx_b,
                    grows_a, grows_b, cnt_ref, sem_a, sem_b, sem_o):
    w = lax.axis_index("s") * 2 + lax.axis_index("c")
    b = w // _W_PER_B
    q = w % _W_PER_B
    pltpu.sync_copy(xyz3_hbm.at[b * 3 + 0, 0], xs)
    pltpu.sync_copy(xyz3_hbm.at[b * 3 + 1, 0], ys)
    pltpu.sync_copy(xyz3_hbm.at[b * 3 + 2, 0], zs)
    pltpu.sync_copy(cen4_hbm.at[w, 0], cenv)
    bn = b * N
    row0 = (b * _SCH + q * _RPW) * NS
    zero16 = jnp.zeros((16,), jnp.int32)
    lane = lax.iota(jnp.int32, 16)

    def scan_row(i, gidx, off):
        cx = cenv[pl.ds(i, 16)][0]
        cy = cenv[pl.ds(_RPW + i, 16)][0]
        cz = cenv[pl.ds(2 * _RPW + i, 16)][0]
        rowbuf[pl.ds(0, 16)] = zero16
        cnt_ref[0] = jnp.int32(0)

        def super_body(sc, carry2):
            t0 = cnt_ref[0]

            @pl.when(t0 < NS)
            def _():
                t = t0
                for u in range(_SUB_PER_SUPER):
                    base = sc * _SUPER + u * 16
                    xv = xs[pl.ds(base, 16)]
                    yv = ys[pl.ds(base, 16)]
                    zv = zs[pl.ds(base, 16)]
                    dx = cx - xv
                    dy = cy - yv
                    dz = cz - zv
                    d2 = dx * dx + dy * dy + dz * dz
                    m = d2 < RR
                    jv = base + lane
                    toff = jnp.minimum(t, NS + 16)
                    plsc.store_compressed(rowbuf.at[pl.ds(toff, 16)], jv, mask=m)
                    t = t + plsc.all_reduce_population_count(m)[0]
                cnt_ref[0] = t

            return carry2

        lax.fori_loop(0, N // _SUPER, super_body, jnp.int32(0))
        t = cnt_ref[0]
        first = rowbuf[pl.ds(0, 16)][0]
        for r in range(NS // 16):
            v = rowbuf[pl.ds(r * 16, 16)]
            kpos = r * 16 + lane
            v = jnp.where(kpos < t, v, first)
            gidx[pl.ds(off + r * 16, 16)] = v + bn

    def quad_body(qq, carry):
        r0 = 4 * qq
        scan_row(r0, gidx_a, 0)
        scan_row(r0 + 1, gidx_a, NS)
        h_a = pltpu.async_copy(ftf_hbm.at[gidx_a], grows_a, sem_a)
        scan_row(r0 + 2, gidx_b, 0)
        scan_row(r0 + 3, gidx_b, NS)
        h_a.wait()
        o_a = pltpu.async_copy(grows_a,
                               out_hbm.at[pl.ds(row0 + r0 * NS, 2 * NS)],
                               sem_o)
        h_b = pltpu.async_copy(ftf_hbm.at[gidx_b], grows_b, sem_b)
        h_b.wait()
        o_b = pltpu.async_copy(grows_b,
                               out_hbm.at[pl.ds(row0 + (r0 + 2) * NS, 2 * NS)],
                               sem_o)
        o_a.wait()
        o_b.wait()
        return carry

    lax.fori_loop(0, _RPW // 4, quad_body, jnp.int32(0))


def _bq_gather(xyz3, cen4, ftf):
    mesh = plsc.VectorSubcoreMesh(core_axis_name="c", subcore_axis_name="s")
    fn = functools.partial(
        pl.kernel,
        out_type=jax.ShapeDtypeStruct((_G_ROWS, CIN), jnp.float32),
        mesh=mesh,
        scratch_types=[
            pltpu.VMEM((N,), jnp.float32),
            pltpu.VMEM((N,), jnp.float32),
            pltpu.VMEM((N,), jnp.float32),
            pltpu.VMEM((128,), jnp.float32),
            pltpu.VMEM((NS + 32,), jnp.int32),
            pltpu.VMEM((2 * NS,), jnp.int32),
            pltpu.VMEM((2 * NS,), jnp.int32),
            pltpu.VMEM((2 * NS, CIN), jnp.float32),
            pltpu.VMEM((2 * NS, CIN), jnp.float32),
            pltpu.SMEM((1,), jnp.int32),
            pltpu.SemaphoreType.DMA,
            pltpu.SemaphoreType.DMA,
            pltpu.SemaphoreType.DMA,
        ],
        compiler_params=pltpu.CompilerParams(needs_layout_passes=False),
    )(_bq_gather_body)
    return fn(xyz3, cen4, ftf)


def _cen4_layout(cen_c):
    t = cen_c.reshape(_W_PER_B, _RPW, B, 3).transpose(2, 0, 3, 1)
    t = t.reshape(B * _W_PER_B, 3 * _RPW)
    t = jnp.pad(t, ((0, 0), (0, 128 - 3 * _RPW)))
    return t.reshape(B * _W_PER_B, 1, 128)


_MM_BLK_S = 32
_MM_ROWS = _MM_BLK_S * NS


def _mm_body(g_ref, w_ref, b_ref, maxes_ref, stats_ref, acc_ref):
    pid = pl.program_id(0)
    y = lax.dot_general(g_ref[...], w_ref[...], (((1,), (1,)), ((), ())),
                        preferred_element_type=jnp.float32)
    y = y + b_ref[...]

    @pl.when(pid == 0)
    def _():
        acc_ref[...] = jnp.zeros_like(acc_ref)

    acc_ref[0, :] += jnp.sum(y, axis=0)
    acc_ref[1, :] += jnp.sum(y * y, axis=0)
    maxes_ref[...] = jnp.max(y.reshape(_MM_BLK_S, NS, COUT), axis=1)

    @pl.when(pid == pl.num_programs(0) - 1)
    def _():
        stats_ref[...] = acc_ref[...]


def _mm(g, W, b2):
    n_prog = _G_ROWS // _MM_ROWS
    return pl.pallas_call(
        _mm_body,
        grid=(n_prog,),
        in_specs=[
            pl.BlockSpec((_MM_ROWS, CIN), lambda i: (i, 0)),
            pl.BlockSpec((COUT, CIN), lambda i: (0, 0)),
            pl.BlockSpec((1, COUT), lambda i: (0, 0)),
        ],
        out_specs=[
            pl.BlockSpec((_MM_BLK_S, COUT), lambda i: (i, 0)),
            pl.BlockSpec((2, COUT), lambda i: (0, 0)),
        ],
        out_shape=[
            jax.ShapeDtypeStruct((B * _SCH, COUT), jnp.float32),
            jax.ShapeDtypeStruct((2, COUT), jnp.float32),
        ],
        scratch_shapes=[pltpu.VMEM((2, COUT), jnp.float32)],
    )(g, W, b2)


def _norm_body(maxes_ref, stats_ref, gamma_ref, beta_ref, out_ref):
    inv_m = jnp.float32(1.0 / M_TOT)
    ssum = (stats_ref[0:1, :] + stats_ref[2:3, :]
            + stats_ref[4:5, :] + stats_ref[6:7, :])
    ssq = (stats_ref[1:2, :] + stats_ref[3:4, :]
           + stats_ref[5:6, :] + stats_ref[7:8, :])
    mean = ssum * inv_m
    var = ssq * inv_m - mean * mean
    rstd = lax.rsqrt(var + EPS)
    o = (maxes_ref[...] - mean) * rstd * gamma_ref[...] + beta_ref[...]
    out_ref[...] = jnp.maximum(o, 0.0)


def _norm(maxes, stats, gamma2, beta2):
    return pl.pallas_call(
        _norm_body,
        out_shape=jax.ShapeDtypeStruct((B * S, COUT), jnp.float32),
    )(maxes, stats, gamma2, beta2)


def kernel(xyz, features, W, b, gamma, beta):
    xyz_sb = jnp.transpose(xyz, (2, 0, 1)).reshape(3 * B, N)
    xyz3 = jnp.transpose(xyz, (0, 2, 1)).reshape(B * 3, 1, N)
    ftf = jnp.transpose(features, (0, 2, 1)).reshape(B * N, CIN)
    b2 = b.reshape(1, COUT)

    dists = jnp.full((B, N), 1e10, jnp.float32)
    far = jnp.zeros((B, 1), jnp.int32)
    xyz3, ftf, dists, far = lax.optimization_barrier((xyz3, ftf, dists, far))
    maxes_parts = []
    stats_parts = []
    for _c in range(_NCH):
        cen_c, dists, far = _fps_chunk(xyz_sb, dists, far)
        cen4_c = _cen4_layout(cen_c)
        cen4_c, dists, far = lax.optimization_barrier((cen4_c, dists, far))
        g_c = _bq_gather(xyz3, cen4_c, ftf)
        mx_c, st_c = _mm(g_c, W, b2)
        maxes_parts.append(mx_c)
        stats_parts.append(st_c)

    maxes = jnp.stack(maxes_parts, 0).reshape(_NCH, B, _SCH, COUT)
    maxes = maxes.transpose(1, 0, 2, 3).reshape(B * S, COUT)
    stats = jnp.concatenate(stats_parts, 0)
    o = _norm(maxes, stats, gamma.reshape(1, COUT), beta.reshape(1, COUT))
    return o.reshape(B, S, COUT).transpose(0, 2, 1)


# speedup vs baseline: 1.0821x; 1.0821x over previous
import functools

import jax
import jax.numpy as jnp
import numpy as np
from jax import lax
from jax.experimental import pallas as pl
from jax.experimental.pallas import tpu as pltpu
from jax.experimental.pallas import tpu_sc as plsc

B = 8
N = 16384
S = 256
NS = 64
CIN = 128
COUT = 256
EPS = 1e-5
RR = np.float32(0.32 * 0.32)
M_TOT = B * S * NS

_NCH = 4
_SCH = S // _NCH
_G_ROWS = B * _SCH * NS


def _fps_body(xyz_sb_ref, dists_in_ref, far_in_ref,
              cen_ref, dists_ref, far_ref):
    i = pl.program_id(0)
    xs = xyz_sb_ref[0:B, :]
    ys = xyz_sb_ref[B:2 * B, :]
    zs = xyz_sb_ref[2 * B:3 * B, :]
    iota = lax.broadcasted_iota(jnp.int32, (B, N), 1)

    @pl.when(i == 0)
    def _():
        dists_ref[...] = dists_in_ref[...]
        far_ref[...] = far_in_ref[...]

    far = far_ref[...]
    onehot = iota == far
    cx = jnp.sum(jnp.where(onehot, xs, 0.0), axis=1, keepdims=True)
    cy = jnp.sum(jnp.where(onehot, ys, 0.0), axis=1, keepdims=True)
    cz = jnp.sum(jnp.where(onehot, zs, 0.0), axis=1, keepdims=True)
    cen_ref[...] = jnp.concatenate([cx, cy, cz], axis=1)[None]
    dx = xs - cx
    dy = ys - cy
    dz = zs - cz
    d = dx * dx + dy * dy + dz * dz
    dmin = jnp.minimum(dists_ref[...], d)
    dists_ref[...] = dmin
    mx = jnp.max(dmin, axis=1, keepdims=True)
    cand = jnp.where(dmin == mx, iota, N)
    far_ref[...] = jnp.min(cand, axis=1, keepdims=True).astype(jnp.int32)


def _fps_chunk(xyz_sb, dists, far):
    return pl.pallas_call(
        _fps_body,
        grid=(_SCH,),
        in_specs=[
            pl.BlockSpec((3 * B, N), lambda i: (0, 0)),
            pl.BlockSpec((B, N), lambda i: (0, 0)),
            pl.BlockSpec((B, 1), lambda i: (0, 0)),
        ],
        out_specs=[
            pl.BlockSpec((1, B, 3), lambda i: (i, 0, 0)),
            pl.BlockSpec((B, N), lambda i: (0, 0)),
            pl.BlockSpec((B, 1), lambda i: (0, 0)),
        ],
        out_shape=[
            jax.ShapeDtypeStruct((_SCH, B, 3), jnp.float32),
            jax.ShapeDtypeStruct((B, N), jnp.float32),
            jax.ShapeDtypeStruct((B, 1), jnp.int32),
        ],
    )(xyz_sb, dists, far)


_RPW = 16
_W_PER_B = 4
_SUB_PER_SUPER = 16
_SUPER = 16 * _SUB_PER_SUPER


def _bq_gather_body(xyz3_hbm, cen4_hbm, ftf_hbm, out_hbm,
                    xs, ys, zs, cenv, rowbuf, gidx_a, gidx_b,
                    grows_a, grows_b, cnt_ref, sem_a, sem_b, sem_o):
    w = lax.axis_index("s") * 2 + lax.axis_index("c")
    b = w // _W_PER_B
    q = w % _W_PER_B
    pltpu.sync_copy(xyz3_hbm.at[b * 3 + 0, 0], xs)
    pltpu.sync_copy(xyz3_hbm.at[b * 3 + 1, 0], ys)
    pltpu.sync_copy(xyz3_hbm.at[b * 3 + 2, 0], zs)
    pltpu.sync_copy(cen4_hbm.at[w, 0], cenv)
    bn = b * N
    row0 = (b * _SCH + q * _RPW) * NS
    zero16 = jnp.zeros((16,), jnp.int32)
    lane = lax.iota(jnp.int32, 16)

    def scan_row(i, gidx, off):
        cx = cenv[pl.ds(i, 16)][0]
        cy = cenv[pl.ds(_RPW + i, 16)][0]
        cz = cenv[pl.ds(2 * _RPW + i, 16)][0]
        rowbuf[pl.ds(0, 16)] = zero16
        cnt_ref[0] = jnp.int32(0)

        def super_body(sc, carry2):
            t0 = cnt_ref[0]

            @pl.when(t0 < NS)
            def _():
                t = t0
                for u in range(_SUB_PER_SUPER):
                    base = sc * _SUPER + u * 16
                    xv = xs[pl.ds(base, 16)]
                    yv = ys[pl.ds(base, 16)]
                    zv = zs[pl.ds(base, 16)]
                    dx = cx - xv
                    dy = cy - yv
                    dz = cz - zv
                    d2 = dx * dx + dy * dy + dz * dz
                    m = d2 < RR
                    jv = base + lane
                    toff = jnp.minimum(t, NS + 16)
                    plsc.store_compressed(rowbuf.at[pl.ds(toff, 16)], jv, mask=m)
                    t = t + plsc.all_reduce_population_count(m)[0]
                cnt_ref[0] = t

            return carry2

        lax.fori_loop(0, N // _SUPER, super_body, jnp.int32(0))
        t = cnt_ref[0]
        first = rowbuf[pl.ds(0, 16)][0]
        for r in range(NS // 16):
            v = rowbuf[pl.ds(r * 16, 16)]
            kpos = r * 16 + lane
            v = jnp.where(kpos < t, v, first)
            gidx[pl.ds(off + r * 16, 16)] = v + bn

    def quad_body(qq, carry):
        r0 = 4 * qq
        scan_row(r0, gidx_a, 0)
        scan_row(r0 + 1, gidx_a, NS)
        h_a = pltpu.async_copy(ftf_hbm.at[gidx_a], grows_a, sem_a)
        scan_row(r0 + 2, gidx_b, 0)
        scan_row(r0 + 3, gidx_b, NS)
        h_a.wait()
        o_a = pltpu.async_copy(grows_a,
                               out_hbm.at[pl.ds(row0 + r0 * NS, 2 * NS)],
                               sem_o)
        h_b = pltpu.async_copy(ftf_hbm.at[gidx_b], grows_b, sem_b)
        h_b.wait()
        o_b = pltpu.async_copy(grows_b,
                               out_hbm.at[pl.ds(row0 + (r0 + 2) * NS, 2 * NS)],
                               sem_o)
        o_a.wait()
        o_b.wait()
        return carry

    lax.fori_loop(0, _RPW // 4, quad_body, jnp.int32(0))


def _bq_gather(xyz3, cen4, ftf):
    mesh = plsc.VectorSubcoreMesh(core_axis_name="c", subcore_axis_name="s")
    fn = functools.partial(
        pl.kernel,
        out_type=jax.ShapeDtypeStruct((_G_ROWS, CIN), jnp.float32),
        mesh=mesh,
        scratch_types=[
            pltpu.VMEM((N,), jnp.float32),
            pltpu.VMEM((N,), jnp.float32),
            pltpu.VMEM((N,), jnp.float32),
            pltpu.VMEM((128,), jnp.float32),
            pltpu.VMEM((NS + 32,), jnp.int32),
            pltpu.VMEM((2 * NS,), jnp.int32),
            pltpu.VMEM((2 * NS,), jnp.int32),
            pltpu.VMEM((2 * NS, CIN), jnp.float32),
            pltpu.VMEM((2 * NS, CIN), jnp.float32),
            pltpu.SMEM((1,), jnp.int32),
            pltpu.SemaphoreType.DMA,
            pltpu.SemaphoreType.DMA,
            pltpu.SemaphoreType.DMA,
        ],
        compiler_params=pltpu.CompilerParams(needs_layout_passes=False),
    )(_bq_gather_body)
    return fn(xyz3, cen4, ftf)


def _cen4_layout(cen_c):
    t = cen_c.reshape(_W_PER_B, _RPW, B, 3).transpose(2, 0, 3, 1)
    t = t.reshape(B * _W_PER_B, 3 * _RPW)
    t = jnp.pad(t, ((0, 0), (0, 128 - 3 * _RPW)))
    return t.reshape(B * _W_PER_B, 1, 128)


_MM_BLK_S = 32
_MM_ROWS = _MM_BLK_S * NS


def _mm_body(g_ref, w_ref, b_ref, maxes_ref, stats_ref, acc_ref):
    pid = pl.program_id(0)
    y = lax.dot_general(g_ref[...], w_ref[...], (((1,), (1,)), ((), ())),
                        preferred_element_type=jnp.float32)
    y = y + b_ref[...]

    @pl.when(pid == 0)
    def _():
        acc_ref[...] = jnp.zeros_like(acc_ref)

    acc_ref[0, :] += jnp.sum(y, axis=0)
    acc_ref[1, :] += jnp.sum(y * y, axis=0)
    maxes_ref[...] = jnp.max(y.reshape(_MM_BLK_S, NS, COUT), axis=1)

    @pl.when(pid == pl.num_programs(0) - 1)
    def _():
        stats_ref[...] = acc_ref[...]


def _mm(g, W, b2):
    n_prog = _G_ROWS // _MM_ROWS
    return pl.pallas_call(
        _mm_body,
        grid=(n_prog,),
        in_specs=[
            pl.BlockSpec((_MM_ROWS, CIN), lambda i: (i, 0)),
            pl.BlockSpec((COUT, CIN), lambda i: (0, 0)),
            pl.BlockSpec((1, COUT), lambda i: (0, 0)),
        ],
        out_specs=[
            pl.BlockSpec((_MM_BLK_S, COUT), lambda i: (i, 0)),
            pl.BlockSpec((2, COUT), lambda i: (0, 0)),
        ],
        out_shape=[
            jax.ShapeDtypeStruct((B * _SCH, COUT), jnp.float32),
            jax.ShapeDtypeStruct((2, COUT), jnp.float32),
        ],
        scratch_shapes=[pltpu.VMEM((2, COUT), jnp.float32)],
    )(g, W, b2)


def _norm_body(maxes_ref, stats_ref, gamma_ref, beta_ref, out_ref):
    inv_m = jnp.float32(1.0 / M_TOT)
    ssum = (stats_ref[0:1, :] + stats_ref[2:3, :]
            + stats_ref[4:5, :] + stats_ref[6:7, :])
    ssq = (stats_ref[1:2, :] + stats_ref[3:4, :]
           + stats_ref[5:6, :] + stats_ref[7:8, :])
    mean = ssum * inv_m
    var = ssq * inv_m - mean * mean
    rstd = lax.rsqrt(var + EPS)
    o = (maxes_ref[...] - mean) * rstd * gamma_ref[...] + beta_ref[...]
    out_ref[...] = jnp.maximum(o, 0.0)


def _norm(maxes, stats, gamma2, beta2):
    return pl.pallas_call(
        _norm_body,
        out_shape=jax.ShapeDtypeStruct((B * S, COUT), jnp.float32),
    )(maxes, stats, gamma2, beta2)


def kernel(xyz, features, W, b, gamma, beta):
    xyz_sb = jnp.transpose(xyz, (2, 0, 1)).reshape(3 * B, N)
    xyz3 = jnp.transpose(xyz, (0, 2, 1)).reshape(B * 3, 1, N)
    ftf = jnp.transpose(features, (0, 2, 1)).reshape(B * N, CIN)
    b2 = b.reshape(1, COUT)

    dists = jnp.full((B, N), 1e10, jnp.float32)
    far = jnp.zeros((B, 1), jnp.int32)
    xyz3, dists, far = lax.optimization_barrier((xyz3, dists, far))
    maxes_parts = []
    stats_parts = []
    for _c in range(_NCH):
        cen_c, dists, far = _fps_chunk(xyz_sb, dists, far)
        cen4_c = _cen4_layout(cen_c)
        cen4_c, dists, far = lax.optimization_barrier((cen4_c, dists, far))
        g_c = _bq_gather(xyz3, cen4_c, ftf)
        mx_c, st_c = _mm(g_c, W, b2)
        maxes_parts.append(mx_c)
        stats_parts.append(st_c)

    maxes = jnp.stack(maxes_parts, 0).reshape(_NCH, B, _SCH, COUT)
    maxes = maxes.transpose(1, 0, 2, 3).reshape(B * S, COUT)
    stats = jnp.concatenate(stats_parts, 0)
    o = _norm(maxes, stats, gamma.reshape(1, COUT), beta.reshape(1, COUT))
    return o.reshape(B, S, COUT).transpose(0, 2, 1)


# speedup vs baseline: 1.0948x; 1.0117x over previous
import functools

import jax
import jax.numpy as jnp
import numpy as np
from jax import lax
from jax.experimental import pallas as pl
from jax.experimental.pallas import tpu as pltpu
from jax.experimental.pallas import tpu_sc as plsc

B = 8
N = 16384
S = 256
NS = 64
CIN = 128
COUT = 256
EPS = 1e-5
RR = np.float32(0.32 * 0.32)
M_TOT = B * S * NS

_NCH = 4
_SCH = S // _NCH
_G_ROWS = B * _SCH * NS


def _fps_body(xyz_sb_ref, dists_in_ref, far_in_ref,
              cen_ref, dists_ref, far_ref):
    i = pl.program_id(0)
    xs = xyz_sb_ref[0:B, :]
    ys = xyz_sb_ref[B:2 * B, :]
    zs = xyz_sb_ref[2 * B:3 * B, :]
    iota = lax.broadcasted_iota(jnp.int32, (B, N), 1)

    @pl.when(i == 0)
    def _():
        dists_ref[...] = dists_in_ref[...]
        far_ref[...] = far_in_ref[...]

    far = far_ref[...]
    onehot = iota == far
    cx = jnp.sum(jnp.where(onehot, xs, 0.0), axis=1, keepdims=True)
    cy = jnp.sum(jnp.where(onehot, ys, 0.0), axis=1, keepdims=True)
    cz = jnp.sum(jnp.where(onehot, zs, 0.0), axis=1, keepdims=True)
    cen_ref[...] = jnp.concatenate([cx, cy, cz], axis=1)[None]
    dx = xs - cx
    dy = ys - cy
    dz = zs - cz
    d = dx * dx + dy * dy + dz * dz
    dmin = jnp.minimum(dists_ref[...], d)
    dists_ref[...] = dmin
    mx = jnp.max(dmin, axis=1, keepdims=True)
    cand = jnp.where(dmin == mx, iota, N)
    far_ref[...] = jnp.min(cand, axis=1, keepdims=True).astype(jnp.int32)


def _fps_chunk(xyz_sb, dists, far):
    return pl.pallas_call(
        _fps_body,
        grid=(_SCH,),
        in_specs=[
            pl.BlockSpec((3 * B, N), lambda i: (0, 0)),
            pl.BlockSpec((B, N), lambda i: (0, 0)),
            pl.BlockSpec((B, 1), lambda i: (0, 0)),
        ],
        out_specs=[
            pl.BlockSpec((1, B, 3), lambda i: (i, 0, 0)),
            pl.BlockSpec((B, N), lambda i: (0, 0)),
            pl.BlockSpec((B, 1), lambda i: (0, 0)),
        ],
        out_shape=[
            jax.ShapeDtypeStruct((_SCH, B, 3), jnp.float32),
            jax.ShapeDtypeStruct((B, N), jnp.float32),
            jax.ShapeDtypeStruct((B, 1), jnp.int32),
        ],
    )(xyz_sb, dists, far)


_RPW = 16
_W_PER_B = 4
_SUB_PER_SUPER = 16
_SUPER = 16 * _SUB_PER_SUPER


def _bq_gather_body(xyz3_hbm, cen4_hbm, ftf_hbm, out_hbm,
                    xs, ys, zs, cenv, rowbuf, gidx_a, gidx_b,
                    grows_a, grows_b, cnt_ref, sem_a, sem_b, sem_o):
    w = lax.axis_index("s") * 2 + lax.axis_index("c")
    b = w // _W_PER_B
    q = w % _W_PER_B
    pltpu.sync_copy(xyz3_hbm.at[b * 3 + 0, 0], xs)
    pltpu.sync_copy(xyz3_hbm.at[b * 3 + 1, 0], ys)
    pltpu.sync_copy(xyz3_hbm.at[b * 3 + 2, 0], zs)
    pltpu.sync_copy(cen4_hbm.at[w, 0], cenv)
    bn = b * N
    row0 = (b * _SCH + q * _RPW) * NS
    zero16 = jnp.zeros((16,), jnp.int32)
    lane = lax.iota(jnp.int32, 16)

    def scan_row(i, gidx, off):
        cx = cenv[pl.ds(i, 16)][0]
        cy = cenv[pl.ds(_RPW + i, 16)][0]
        cz = cenv[pl.ds(2 * _RPW + i, 16)][0]
        rowbuf[pl.ds(0, 16)] = zero16
        cnt_ref[0] = jnp.int32(0)

        def super_body(sc, carry2):
            t0 = cnt_ref[0]

            @pl.when(t0 < NS)
            def _():
                t = t0
                for u in range(_SUB_PER_SUPER):
                    base = sc * _SUPER + u * 16
                    xv = xs[pl.ds(base, 16)]
                    yv = ys[pl.ds(base, 16)]
                    zv = zs[pl.ds(base, 16)]
                    dx = cx - xv
                    dy = cy - yv
                    dz = cz - zv
                    d2 = dx * dx + dy * dy + dz * dz
                    m = d2 < RR
                    jv = base + lane
                    toff = jnp.minimum(t, NS + 16)
                    plsc.store_compressed(rowbuf.at[pl.ds(toff, 16)], jv, mask=m)
                    t = t + plsc.all_reduce_population_count(m)[0]
                cnt_ref[0] = t

            return carry2

        lax.fori_loop(0, N // _SUPER, super_body, jnp.int32(0))
        t = cnt_ref[0]
        first = rowbuf[pl.ds(0, 16)][0]
        for r in range(NS // 16):
            v = rowbuf[pl.ds(r * 16, 16)]
            kpos = r * 16 + lane
            v = jnp.where(kpos < t, v, first)
            gidx[pl.ds(off + r * 16, 16)] = v + bn

    def quad_body(qq, carry):
        r0 = 4 * qq
        scan_row(r0, gidx_a, 0)
        scan_row(r0 + 1, gidx_a, NS)
        h_a = pltpu.async_copy(ftf_hbm.at[gidx_a], grows_a, sem_a)
        scan_row(r0 + 2, gidx_b, 0)
        scan_row(r0 + 3, gidx_b, NS)
        h_b = pltpu.async_copy(ftf_hbm.at[gidx_b], grows_b, sem_b)
        h_a.wait()
        o_a = pltpu.async_copy(grows_a,
                               out_hbm.at[pl.ds(row0 + r0 * NS, 2 * NS)],
                               sem_o)
        h_b.wait()
        o_b = pltpu.async_copy(grows_b,
                               out_hbm.at[pl.ds(row0 + (r0 + 2) * NS, 2 * NS)],
                               sem_o)
        o_a.wait()
        o_b.wait()
        return carry

    lax.fori_loop(0, _RPW // 4, quad_body, jnp.int32(0))


def _bq_gather(xyz3, cen4, ftf):
    mesh = plsc.VectorSubcoreMesh(core_axis_name="c", subcore_axis_name="s")
    fn = functools.partial(
        pl.kernel,
        out_type=jax.ShapeDtypeStruct((_G_ROWS, CIN), jnp.float32),
        mesh=mesh,
        scratch_types=[
            pltpu.VMEM((N,), jnp.float32),
            pltpu.VMEM((N,), jnp.float32),
            pltpu.VMEM((N,), jnp.float32),
            pltpu.VMEM((128,), jnp.float32),
            pltpu.VMEM((NS + 32,), jnp.int32),
            pltpu.VMEM((2 * NS,), jnp.int32),
            pltpu.VMEM((2 * NS,), jnp.int32),
            pltpu.VMEM((2 * NS, CIN), jnp.float32),
            pltpu.VMEM((2 * NS, CIN), jnp.float32),
            pltpu.SMEM((1,), jnp.int32),
            pltpu.SemaphoreType.DMA,
            pltpu.SemaphoreType.DMA,
            pltpu.SemaphoreType.DMA,
        ],
        compiler_params=pltpu.CompilerParams(needs_layout_passes=False),
    )(_bq_gather_body)
    return fn(xyz3, cen4, ftf)


def _cen4_layout(cen_c):
    t = cen_c.reshape(_W_PER_B, _RPW, B, 3).transpose(2, 0, 3, 1)
    t = t.reshape(B * _W_PER_B, 3 * _RPW)
    t = jnp.pad(t, ((0, 0), (0, 128 - 3 * _RPW)))
    return t.reshape(B * _W_PER_B, 1, 128)


_MM_BLK_S = 32
_MM_ROWS = _MM_BLK_S * NS


def _mm_body(g_ref, w_ref, b_ref, maxes_ref, stats_ref, acc_ref):
    pid = pl.program_id(0)
    y = lax.dot_general(g_ref[...], w_ref[...], (((1,), (1,)), ((), ())),
                        preferred_element_type=jnp.float32)
    y = y + b_ref[...]

    @pl.when(pid == 0)
    def _():
        acc_ref[...] = jnp.zeros_like(acc_ref)

    acc_ref[0, :] += jnp.sum(y, axis=0)
    acc_ref[1, :] += jnp.sum(y * y, axis=0)
    maxes_ref[...] = jnp.max(y.reshape(_MM_BLK_S, NS, COUT), axis=1)

    @pl.when(pid == pl.num_programs(0) - 1)
    def _():
        stats_ref[...] = acc_ref[...]


def _mm(g, W, b2):
    n_prog = _G_ROWS // _MM_ROWS
    return pl.pallas_call(
        _mm_body,
        grid=(n_prog,),
        in_specs=[
            pl.BlockSpec((_MM_ROWS, CIN), lambda i: (i, 0)),
            pl.BlockSpec((COUT, CIN), lambda i: (0, 0)),
            pl.BlockSpec((1, COUT), lambda i: (0, 0)),
        ],
        out_specs=[
            pl.BlockSpec((_MM_BLK_S, COUT), lambda i: (i, 0)),
            pl.BlockSpec((2, COUT), lambda i: (0, 0)),
        ],
        out_shape=[
            jax.ShapeDtypeStruct((B * _SCH, COUT), jnp.float32),
            jax.ShapeDtypeStruct((2, COUT), jnp.float32),
        ],
        scratch_shapes=[pltpu.VMEM((2, COUT), jnp.float32)],
    )(g, W, b2)


def _norm_body(maxes_ref, stats_ref, gamma_ref, beta_ref, out_ref):
    inv_m = jnp.float32(1.0 / M_TOT)
    ssum = (stats_ref[0:1, :] + stats_ref[2:3, :]
            + stats_ref[4:5, :] + stats_ref[6:7, :])
    ssq = (stats_ref[1:2, :] + stats_ref[3:4, :]
           + stats_ref[5:6, :] + stats_ref[7:8, :])
    mean = ssum * inv_m
    var = ssq * inv_m - mean * mean
    rstd = lax.rsqrt(var + EPS)
    o = (maxes_ref[...] - mean) * rstd * gamma_ref[...] + beta_ref[...]
    out_ref[...] = jnp.maximum(o, 0.0)


def _norm(maxes, stats, gamma2, beta2):
    return pl.pallas_call(
        _norm_body,
        out_shape=jax.ShapeDtypeStruct((B * S, COUT), jnp.float32),
    )(maxes, stats, gamma2, beta2)


def kernel(xyz, features, W, b, gamma, beta):
    xyz_sb = jnp.transpose(xyz, (2, 0, 1)).reshape(3 * B, N)
    xyz3 = jnp.transpose(xyz, (0, 2, 1)).reshape(B * 3, 1, N)
    ftf = jnp.transpose(features, (0, 2, 1)).reshape(B * N, CIN)
    b2 = b.reshape(1, COUT)

    dists = jnp.full((B, N), 1e10, jnp.float32)
    far = jnp.zeros((B, 1), jnp.int32)
    maxes_parts = []
    stats_parts = []
    for _c in range(_NCH):
        cen_c, dists, far = _fps_chunk(xyz_sb, dists, far)
        g_c = _bq_gather(xyz3, _cen4_layout(cen_c), ftf)
        mx_c, st_c = _mm(g_c, W, b2)
        maxes_parts.append(mx_c)
        stats_parts.append(st_c)

    maxes = jnp.stack(maxes_parts, 0).reshape(_NCH, B, _SCH, COUT)
    maxes = maxes.transpose(1, 0, 2, 3).reshape(B * S, COUT)
    stats = jnp.concatenate(stats_parts, 0)
    o = _norm(maxes, stats, gamma.reshape(1, COUT), beta.reshape(1, COUT))
    return o.reshape(B, S, COUT).transpose(0, 2, 1)


# speedup vs baseline: 1.1814x; 1.0791x over previous
import functools

import jax
import jax.numpy as jnp
import numpy as np
from jax import lax
from jax.experimental import pallas as pl
from jax.experimental.pallas import tpu as pltpu
from jax.experimental.pallas import tpu_sc as plsc

B = 8
N = 16384
S = 256
NS = 64
CIN = 128
COUT = 256
EPS = 1e-5
RR = np.float32(0.32 * 0.32)
M_TOT = B * S * NS

_NCH = 4
_SCH = S // _NCH
_G_ROWS = B * _SCH * NS


_FPS_UNROLL = 4


def _fps_body(xyz_sb_ref, dists_in_ref, far_in_ref,
              cen_ref, dists_ref, far_ref):
    i = pl.program_id(0)
    xs = xyz_sb_ref[0:B, :]
    ys = xyz_sb_ref[B:2 * B, :]
    zs = xyz_sb_ref[2 * B:3 * B, :]
    iota = lax.broadcasted_iota(jnp.int32, (B, N), 1)

    @pl.when(i == 0)
    def _():
        dists_ref[...] = dists_in_ref[...]
        far_ref[...] = far_in_ref[...]

    far = far_ref[...]
    for k in range(_FPS_UNROLL):
        onehot = iota == far
        cx = jnp.sum(jnp.where(onehot, xs, 0.0), axis=1, keepdims=True)
        cy = jnp.sum(jnp.where(onehot, ys, 0.0), axis=1, keepdims=True)
        cz = jnp.sum(jnp.where(onehot, zs, 0.0), axis=1, keepdims=True)
        cen_ref[k] = jnp.concatenate([cx, cy, cz], axis=1)
        dx = xs - cx
        dy = ys - cy
        dz = zs - cz
        d = dx * dx + dy * dy + dz * dz
        dmin = jnp.minimum(dists_ref[...], d)
        dists_ref[...] = dmin
        mx = jnp.max(dmin, axis=1, keepdims=True)
        cand = jnp.where(dmin == mx, iota, N)
        far = jnp.min(cand, axis=1, keepdims=True).astype(jnp.int32)
    far_ref[...] = far


def _fps_chunk(xyz_sb, dists, far):
    return pl.pallas_call(
        _fps_body,
        grid=(_SCH // _FPS_UNROLL,),
        in_specs=[
            pl.BlockSpec((3 * B, N), lambda i: (0, 0)),
            pl.BlockSpec((B, N), lambda i: (0, 0)),
            pl.BlockSpec((B, 1), lambda i: (0, 0)),
        ],
        out_specs=[
            pl.BlockSpec((_FPS_UNROLL, B, 3), lambda i: (i, 0, 0)),
            pl.BlockSpec((B, N), lambda i: (0, 0)),
            pl.BlockSpec((B, 1), lambda i: (0, 0)),
        ],
        out_shape=[
            jax.ShapeDtypeStruct((_SCH, B, 3), jnp.float32),
            jax.ShapeDtypeStruct((B, N), jnp.float32),
            jax.ShapeDtypeStruct((B, 1), jnp.int32),
        ],
    )(xyz_sb, dists, far)


_RPW = 16
_W_PER_B = 4
_SUB_PER_SUPER = 16
_SUPER = 16 * _SUB_PER_SUPER


def _bq_gather_body(xyz3_hbm, cen4_hbm, ftf_hbm, out_hbm,
                    xs, ys, zs, cenv, rowbuf, gidx_a, gidx_b,
                    grows_a, grows_b, cnt_ref, sem_a, sem_b, sem_o):
    w = lax.axis_index("s") * 2 + lax.axis_index("c")
    b = w // _W_PER_B
    q = w % _W_PER_B
    pltpu.sync_copy(xyz3_hbm.at[b * 3 + 0, 0], xs)
    pltpu.sync_copy(xyz3_hbm.at[b * 3 + 1, 0], ys)
    pltpu.sync_copy(xyz3_hbm.at[b * 3 + 2, 0], zs)
    pltpu.sync_copy(cen4_hbm.at[w, 0], cenv)
    bn = b * N
    row0 = (b * _SCH + q * _RPW) * NS
    zero16 = jnp.zeros((16,), jnp.int32)
    lane = lax.iota(jnp.int32, 16)

    def scan_row(i, gidx, off):
        cx = cenv[pl.ds(i, 16)][0]
        cy = cenv[pl.ds(_RPW + i, 16)][0]
        cz = cenv[pl.ds(2 * _RPW + i, 16)][0]
        rowbuf[pl.ds(0, 16)] = zero16
        cnt_ref[0] = jnp.int32(0)

        def super_body(sc, carry2):
            t0 = cnt_ref[0]

            @pl.when(t0 < NS)
            def _():
                t = t0
                for u in range(_SUB_PER_SUPER):
                    base = sc * _SUPER + u * 16
                    xv = xs[pl.ds(base, 16)]
                    yv = ys[pl.ds(base, 16)]
                    zv = zs[pl.ds(base, 16)]
                    dx = cx - xv
                    dy = cy - yv
                    dz = cz - zv
                    d2 = dx * dx + dy * dy + dz * dz
                    m = d2 < RR
                    jv = base + lane
                    toff = jnp.minimum(t, NS + 16)
                    plsc.store_compressed(rowbuf.at[pl.ds(toff, 16)], jv, mask=m)
                    t = t + plsc.all_reduce_population_count(m)[0]
                cnt_ref[0] = t

            return carry2

        lax.fori_loop(0, N // _SUPER, super_body, jnp.int32(0))
        t = cnt_ref[0]
        first = rowbuf[pl.ds(0, 16)][0]
        for r in range(NS // 16):
            v = rowbuf[pl.ds(r * 16, 16)]
            kpos = r * 16 + lane
            v = jnp.where(kpos < t, v, first)
            gidx[pl.ds(off + r * 16, 16)] = v + bn

    def quad_body(qq, carry):
        r0 = 4 * qq
        scan_row(r0, gidx_a, 0)
        scan_row(r0 + 1, gidx_a, NS)
        h_a = pltpu.async_copy(ftf_hbm.at[gidx_a], grows_a, sem_a)
        scan_row(r0 + 2, gidx_b, 0)
        scan_row(r0 + 3, gidx_b, NS)
        h_b = pltpu.async_copy(ftf_hbm.at[gidx_b], grows_b, sem_b)
        h_a.wait()
        o_a = pltpu.async_copy(grows_a,
                               out_hbm.at[pl.ds(row0 + r0 * NS, 2 * NS)],
                               sem_o)
        h_b.wait()
        o_b = pltpu.async_copy(grows_b,
                               out_hbm.at[pl.ds(row0 + (r0 + 2) * NS, 2 * NS)],
                               sem_o)
        o_a.wait()
        o_b.wait()
        return carry

    lax.fori_loop(0, _RPW // 4, quad_body, jnp.int32(0))


def _bq_gather(xyz3, cen4, ftf):
    mesh = plsc.VectorSubcoreMesh(core_axis_name="c", subcore_axis_name="s")
    fn = functools.partial(
        pl.kernel,
        out_type=jax.ShapeDtypeStruct((_G_ROWS, CIN), jnp.float32),
        mesh=mesh,
        scratch_types=[
            pltpu.VMEM((N,), jnp.float32),
            pltpu.VMEM((N,), jnp.float32),
            pltpu.VMEM((N,), jnp.float32),
            pltpu.VMEM((128,), jnp.float32),
            pltpu.VMEM((NS + 32,), jnp.int32),
            pltpu.VMEM((2 * NS,), jnp.int32),
            pltpu.VMEM((2 * NS,), jnp.int32),
            pltpu.VMEM((2 * NS, CIN), jnp.float32),
            pltpu.VMEM((2 * NS, CIN), jnp.float32),
            pltpu.SMEM((1,), jnp.int32),
            pltpu.SemaphoreType.DMA,
            pltpu.SemaphoreType.DMA,
            pltpu.SemaphoreType.DMA,
        ],
        compiler_params=pltpu.CompilerParams(needs_layout_passes=False),
    )(_bq_gather_body)
    return fn(xyz3, cen4, ftf)


def _cen4_layout(cen_c):
    t = cen_c.reshape(_W_PER_B, _RPW, B, 3).transpose(2, 0, 3, 1)
    t = t.reshape(B * _W_PER_B, 3 * _RPW)
    t = jnp.pad(t, ((0, 0), (0, 128 - 3 * _RPW)))
    return t.reshape(B * _W_PER_B, 1, 128)


_MM_BLK_S = 32
_MM_ROWS = _MM_BLK_S * NS


def _mm_body(g_ref, w_ref, b_ref, maxes_ref, stats_ref, acc_ref):
    pid = pl.program_id(0)
    y = lax.dot_general(g_ref[...], w_ref[...], (((1,), (1,)), ((), ())),
                        preferred_element_type=jnp.float32)
    y = y + b_ref[...]

    @pl.when(pid == 0)
    def _():
        acc_ref[...] = jnp.zeros_like(acc_ref)

    acc_ref[0, :] += jnp.sum(y, axis=0)
    acc_ref[1, :] += jnp.sum(y * y, axis=0)
    maxes_ref[...] = jnp.max(y.reshape(_MM_BLK_S, NS, COUT), axis=1)

    @pl.when(pid == pl.num_programs(0) - 1)
    def _():
        stats_ref[...] = acc_ref[...]


def _mm(g, W, b2):
    n_prog = _G_ROWS // _MM_ROWS
    return pl.pallas_call(
        _mm_body,
        grid=(n_prog,),
        in_specs=[
            pl.BlockSpec((_MM_ROWS, CIN), lambda i: (i, 0)),
            pl.BlockSpec((COUT, CIN), lambda i: (0, 0)),
            pl.BlockSpec((1, COUT), lambda i: (0, 0)),
        ],
        out_specs=[
            pl.BlockSpec((_MM_BLK_S, COUT), lambda i: (i, 0)),
            pl.BlockSpec((2, COUT), lambda i: (0, 0)),
        ],
        out_shape=[
            jax.ShapeDtypeStruct((B * _SCH, COUT), jnp.float32),
            jax.ShapeDtypeStruct((2, COUT), jnp.float32),
        ],
        scratch_shapes=[pltpu.VMEM((2, COUT), jnp.float32)],
    )(g, W, b2)


def _norm_body(maxes_ref, stats_ref, gamma_ref, beta_ref, out_ref):
    inv_m = jnp.float32(1.0 / M_TOT)
    ssum = (stats_ref[0:1, :] + stats_ref[2:3, :]
            + stats_ref[4:5, :] + stats_ref[6:7, :])
    ssq = (stats_ref[1:2, :] + stats_ref[3:4, :]
           + stats_ref[5:6, :] + stats_ref[7:8, :])
    mean = ssum * inv_m
    var = ssq * inv_m - mean * mean
    rstd = lax.rsqrt(var + EPS)
    o = (maxes_ref[...] - mean) * rstd * gamma_ref[...] + beta_ref[...]
    out_ref[...] = jnp.maximum(o, 0.0)


def _norm(maxes, stats, gamma2, beta2):
    return pl.pallas_call(
        _norm_body,
        out_shape=jax.ShapeDtypeStruct((B * S, COUT), jnp.float32),
    )(maxes, stats, gamma2, beta2)


def kernel(xyz, features, W, b, gamma, beta):
    xyz_sb = jnp.transpose(xyz, (2, 0, 1)).reshape(3 * B, N)
    xyz3 = jnp.transpose(xyz, (0, 2, 1)).reshape(B * 3, 1, N)
    ftf = jnp.transpose(features, (0, 2, 1)).reshape(B * N, CIN)
    b2 = b.reshape(1, COUT)

    dists = jnp.full((B, N), 1e10, jnp.float32)
    far = jnp.zeros((B, 1), jnp.int32)
    maxes_parts = []
    stats_parts = []
    for _c in range(_NCH):
        cen_c, dists, far = _fps_chunk(xyz_sb, dists, far)
        g_c = _bq_gather(xyz3, _cen4_layout(cen_c), ftf)
        mx_c, st_c = _mm(g_c, W, b2)
        maxes_parts.append(mx_c)
        stats_parts.append(st_c)

    maxes = jnp.stack(maxes_parts, 0).reshape(_NCH, B, _SCH, COUT)
    maxes = maxes.transpose(1, 0, 2, 3).reshape(B * S, COUT)
    stats = jnp.concatenate(stats_parts, 0)
    o = _norm(maxes, stats, gamma.reshape(1, COUT), beta.reshape(1, COUT))
    return o.reshape(B, S, COUT).transpose(0, 2, 1)


# speedup vs baseline: 1.2271x; 1.0387x over previous
import functools

import jax
import jax.numpy as jnp
import numpy as np
from jax import lax
from jax.experimental import pallas as pl
from jax.experimental.pallas import tpu as pltpu
from jax.experimental.pallas import tpu_sc as plsc

B = 8
N = 16384
S = 256
NS = 64
CIN = 128
COUT = 256
EPS = 1e-5
RR = np.float32(0.32 * 0.32)
M_TOT = B * S * NS

_NCH = 4
_SCH = S // _NCH
_G_ROWS = B * _SCH * NS


_FPS_UNROLL = 8


def _fps_body(xyz_sb_ref, dists_in_ref, far_in_ref,
              cen_ref, dists_ref, far_ref):
    i = pl.program_id(0)
    xs = xyz_sb_ref[0:B, :]
    ys = xyz_sb_ref[B:2 * B, :]
    zs = xyz_sb_ref[2 * B:3 * B, :]
    iota = lax.broadcasted_iota(jnp.int32, (B, N), 1)

    @pl.when(i == 0)
    def _():
        dists_ref[...] = dists_in_ref[...]
        far_ref[...] = far_in_ref[...]

    far = far_ref[...]
    for k in range(_FPS_UNROLL):
        onehot = iota == far
        cx = jnp.sum(jnp.where(onehot, xs, 0.0), axis=1, keepdims=True)
        cy = jnp.sum(jnp.where(onehot, ys, 0.0), axis=1, keepdims=True)
        cz = jnp.sum(jnp.where(onehot, zs, 0.0), axis=1, keepdims=True)
        cen_ref[k] = jnp.concatenate([cx, cy, cz], axis=1)
        dx = xs - cx
        dy = ys - cy
        dz = zs - cz
        d = dx * dx + dy * dy + dz * dz
        dmin = jnp.minimum(dists_ref[...], d)
        dists_ref[...] = dmin
        mx = jnp.max(dmin, axis=1, keepdims=True)
        cand = jnp.where(dmin == mx, iota, N)
        far = jnp.min(cand, axis=1, keepdims=True).astype(jnp.int32)
    far_ref[...] = far


def _fps_chunk(xyz_sb, dists, far):
    return pl.pallas_call(
        _fps_body,
        grid=(_SCH // _FPS_UNROLL,),
        in_specs=[
            pl.BlockSpec((3 * B, N), lambda i: (0, 0)),
            pl.BlockSpec((B, N), lambda i: (0, 0)),
            pl.BlockSpec((B, 1), lambda i: (0, 0)),
        ],
        out_specs=[
            pl.BlockSpec((_FPS_UNROLL, B, 3), lambda i: (i, 0, 0)),
            pl.BlockSpec((B, N), lambda i: (0, 0)),
            pl.BlockSpec((B, 1), lambda i: (0, 0)),
        ],
        out_shape=[
            jax.ShapeDtypeStruct((_SCH, B, 3), jnp.float32),
            jax.ShapeDtypeStruct((B, N), jnp.float32),
            jax.ShapeDtypeStruct((B, 1), jnp.int32),
        ],
    )(xyz_sb, dists, far)


_RPW = 16
_W_PER_B = 4
_SUB_PER_SUPER = 16
_SUPER = 16 * _SUB_PER_SUPER


def _bq_gather_body(xyz3_hbm, cen4_hbm, ftf_hbm, out_hbm,
                    xs, ys, zs, cenv, rowbuf, gidx_a, gidx_b,
                    grows_a, grows_b, cnt_ref, sem_a, sem_b, sem_o):
    w = lax.axis_index("s") * 2 + lax.axis_index("c")
    b = w // _W_PER_B
    q = w % _W_PER_B
    pltpu.sync_copy(xyz3_hbm.at[b * 3 + 0, 0], xs)
    pltpu.sync_copy(xyz3_hbm.at[b * 3 + 1, 0], ys)
    pltpu.sync_copy(xyz3_hbm.at[b * 3 + 2, 0], zs)
    pltpu.sync_copy(cen4_hbm.at[w, 0], cenv)
    bn = b * N
    row0 = (b * _SCH + q * _RPW) * NS
    zero16 = jnp.zeros((16,), jnp.int32)
    lane = lax.iota(jnp.int32, 16)

    def scan_row(i, gidx, off):
        cx = cenv[pl.ds(i, 16)][0]
        cy = cenv[pl.ds(_RPW + i, 16)][0]
        cz = cenv[pl.ds(2 * _RPW + i, 16)][0]
        rowbuf[pl.ds(0, 16)] = zero16
        cnt_ref[0] = jnp.int32(0)

        def super_body(sc, carry2):
            t0 = cnt_ref[0]

            @pl.when(t0 < NS)
            def _():
                t = t0
                for u in range(_SUB_PER_SUPER):
                    base = sc * _SUPER + u * 16
                    xv = xs[pl.ds(base, 16)]
                    yv = ys[pl.ds(base, 16)]
                    zv = zs[pl.ds(base, 16)]
                    dx = cx - xv
                    dy = cy - yv
                    dz = cz - zv
                    d2 = dx * dx + dy * dy + dz * dz
                    m = d2 < RR
                    jv = base + lane
                    toff = jnp.minimum(t, NS + 16)
                    plsc.store_compressed(rowbuf.at[pl.ds(toff, 16)], jv, mask=m)
                    t = t + plsc.all_reduce_population_count(m)[0]
                cnt_ref[0] = t

            return carry2

        lax.fori_loop(0, N // _SUPER, super_body, jnp.int32(0))
        t = cnt_ref[0]
        first = rowbuf[pl.ds(0, 16)][0]
        for r in range(NS // 16):
            v = rowbuf[pl.ds(r * 16, 16)]
            kpos = r * 16 + lane
            v = jnp.where(kpos < t, v, first)
            gidx[pl.ds(off + r * 16, 16)] = v + bn

    def quad_body(qq, carry):
        r0 = 4 * qq
        scan_row(r0, gidx_a, 0)
        scan_row(r0 + 1, gidx_a, NS)
        h_a = pltpu.async_copy(ftf_hbm.at[gidx_a], grows_a, sem_a)
        scan_row(r0 + 2, gidx_b, 0)
        scan_row(r0 + 3, gidx_b, NS)
        h_b = pltpu.async_copy(ftf_hbm.at[gidx_b], grows_b, sem_b)
        h_a.wait()
        o_a = pltpu.async_copy(grows_a,
                               out_hbm.at[pl.ds(row0 + r0 * NS, 2 * NS)],
                               sem_o)
        h_b.wait()
        o_b = pltpu.async_copy(grows_b,
                               out_hbm.at[pl.ds(row0 + (r0 + 2) * NS, 2 * NS)],
                               sem_o)
        o_a.wait()
        o_b.wait()
        return carry

    lax.fori_loop(0, _RPW // 4, quad_body, jnp.int32(0))


def _bq_gather(xyz3, cen4, ftf):
    mesh = plsc.VectorSubcoreMesh(core_axis_name="c", subcore_axis_name="s")
    fn = functools.partial(
        pl.kernel,
        out_type=jax.ShapeDtypeStruct((_G_ROWS, CIN), jnp.float32),
        mesh=mesh,
        scratch_types=[
            pltpu.VMEM((N,), jnp.float32),
            pltpu.VMEM((N,), jnp.float32),
            pltpu.VMEM((N,), jnp.float32),
            pltpu.VMEM((128,), jnp.float32),
            pltpu.VMEM((NS + 32,), jnp.int32),
            pltpu.VMEM((2 * NS,), jnp.int32),
            pltpu.VMEM((2 * NS,), jnp.int32),
            pltpu.VMEM((2 * NS, CIN), jnp.float32),
            pltpu.VMEM((2 * NS, CIN), jnp.float32),
            pltpu.SMEM((1,), jnp.int32),
            pltpu.SemaphoreType.DMA,
            pltpu.SemaphoreType.DMA,
            pltpu.SemaphoreType.DMA,
        ],
        compiler_params=pltpu.CompilerParams(needs_layout_passes=False),
    )(_bq_gather_body)
    return fn(xyz3, cen4, ftf)


def _cen4_layout(cen_c):
    t = cen_c.reshape(_W_PER_B, _RPW, B, 3).transpose(2, 0, 3, 1)
    t = t.reshape(B * _W_PER_B, 3 * _RPW)
    t = jnp.pad(t, ((0, 0), (0, 128 - 3 * _RPW)))
    return t.reshape(B * _W_PER_B, 1, 128)


_MM_BLK_S = 32
_MM_ROWS = _MM_BLK_S * NS


def _mm_body(g_ref, w_ref, b_ref, maxes_ref, stats_ref, acc_ref):
    pid = pl.program_id(0)
    y = lax.dot_general(g_ref[...], w_ref[...], (((1,), (1,)), ((), ())),
                        preferred_element_type=jnp.float32)
    y = y + b_ref[...]

    @pl.when(pid == 0)
    def _():
        acc_ref[...] = jnp.zeros_like(acc_ref)

    acc_ref[0, :] += jnp.sum(y, axis=0)
    acc_ref[1, :] += jnp.sum(y * y, axis=0)
    maxes_ref[...] = jnp.max(y.reshape(_MM_BLK_S, NS, COUT), axis=1)

    @pl.when(pid == pl.num_programs(0) - 1)
    def _():
        stats_ref[...] = acc_ref[...]


def _mm(g, W, b2):
    n_prog = _G_ROWS // _MM_ROWS
    return pl.pallas_call(
        _mm_body,
        grid=(n_prog,),
        in_specs=[
            pl.BlockSpec((_MM_ROWS, CIN), lambda i: (i, 0)),
            pl.BlockSpec((COUT, CIN), lambda i: (0, 0)),
            pl.BlockSpec((1, COUT), lambda i: (0, 0)),
        ],
        out_specs=[
            pl.BlockSpec((_MM_BLK_S, COUT), lambda i: (i, 0)),
            pl.BlockSpec((2, COUT), lambda i: (0, 0)),
        ],
        out_shape=[
            jax.ShapeDtypeStruct((B * _SCH, COUT), jnp.float32),
            jax.ShapeDtypeStruct((2, COUT), jnp.float32),
        ],
        scratch_shapes=[pltpu.VMEM((2, COUT), jnp.float32)],
    )(g, W, b2)


def _norm_body(maxes_ref, stats_ref, gamma_ref, beta_ref, out_ref):
    inv_m = jnp.float32(1.0 / M_TOT)
    ssum = (stats_ref[0:1, :] + stats_ref[2:3, :]
            + stats_ref[4:5, :] + stats_ref[6:7, :])
    ssq = (stats_ref[1:2, :] + stats_ref[3:4, :]
           + stats_ref[5:6, :] + stats_ref[7:8, :])
    mean = ssum * inv_m
    var = ssq * inv_m - mean * mean
    rstd = lax.rsqrt(var + EPS)
    o = (maxes_ref[...] - mean) * rstd * gamma_ref[...] + beta_ref[...]
    out_ref[...] = jnp.maximum(o, 0.0)


def _norm(maxes, stats, gamma2, beta2):
    return pl.pallas_call(
        _norm_body,
        out_shape=jax.ShapeDtypeStruct((B * S, COUT), jnp.float32),
    )(maxes, stats, gamma2, beta2)


def kernel(xyz, features, W, b, gamma, beta):
    xyz_sb = jnp.transpose(xyz, (2, 0, 1)).reshape(3 * B, N)
    xyz3 = jnp.transpose(xyz, (0, 2, 1)).reshape(B * 3, 1, N)
    ftf = jnp.transpose(features, (0, 2, 1)).reshape(B * N, CIN)
    b2 = b.reshape(1, COUT)

    dists = jnp.full((B, N), 1e10, jnp.float32)
    far = jnp.zeros((B, 1), jnp.int32)
    maxes_parts = []
    stats_parts = []
    for _c in range(_NCH):
        cen_c, dists, far = _fps_chunk(xyz_sb, dists, far)
        g_c = _bq_gather(xyz3, _cen4_layout(cen_c), ftf)
        mx_c, st_c = _mm(g_c, W, b2)
        maxes_parts.append(mx_c)
        stats_parts.append(st_c)

    maxes = jnp.stack(maxes_parts, 0).reshape(_NCH, B, _SCH, COUT)
    maxes = maxes.transpose(1, 0, 2, 3).reshape(B * S, COUT)
    stats = jnp.concatenate(stats_parts, 0)
    o = _norm(maxes, stats, gamma.reshape(1, COUT), beta.reshape(1, COUT))
    return o.reshape(B, S, COUT).transpose(0, 2, 1)
